# baseline (speedup 1.0000x reference)
import functools

import jax
import jax.numpy as jnp
from jax.experimental import pallas as pl
from jax.experimental.pallas import tpu as pltpu

ROWS = 8
CW = 1024


def _sampler_body(nchunks, vsize, p_ref, rng_ref, out_ref, cand_ref):
    rngv = rng_ref[0]
    carry = jnp.zeros((ROWS, 1), jnp.float32)
    base = jnp.zeros((ROWS, 1), jnp.int32)
    pstar = jnp.zeros((ROWS, 1), jnp.float32)
    cwidth = jnp.zeros((ROWS, 1), jnp.int32)

    for c in range(nchunks):
        w = min(CW, vsize - c * CW)
        chunk = p_ref[:, c * CW:c * CW + w]
        s = jnp.sum(chunk, axis=1, keepdims=True)
        new_carry = carry + s
        below = new_carry < rngv
        here = jnp.logical_and(carry < rngv, jnp.logical_not(below))
        base = base + jnp.where(below, w, 0)
        pstar = jnp.where(here, carry, pstar)
        cwidth = jnp.where(here, w, cwidth)

        @pl.when(jnp.any(here))
        def _():
            cand_ref[:, :w] = jnp.where(here, chunk, cand_ref[:, :w])

        carry = new_carry

    cand = cand_ref[...]
    lanes = jax.lax.broadcasted_iota(jnp.int32, (ROWS, CW), 1)
    lc = cand
    sh = 1
    while sh < CW:
        rolled = pltpu.roll(lc, sh, axis=1)
        lc = lc + jnp.where(lanes >= sh, rolled, 0.0)
        sh *= 2
    valid = lanes < cwidth
    cnt = jnp.sum(
        jnp.where(jnp.logical_and(valid, pstar + lc < rngv), 1, 0),
        axis=1, keepdims=True)
    out_ref[0] = base + cnt


@jax.jit
def kernel(p, rng):
    B, T, V = p.shape
    R = B * T
    nchunks = -(-V // CW)
    p2 = p.reshape(R, V)
    rng3 = rng.reshape(R // ROWS, ROWS, 1)

    out = pl.pallas_call(
        functools.partial(_sampler_body, nchunks, V),
        grid=(R // ROWS,),
        in_specs=[
            pl.BlockSpec((ROWS, V), lambda i: (i, 0)),
            pl.BlockSpec((1, ROWS, 1), lambda i: (i, 0, 0)),
        ],
        out_specs=pl.BlockSpec((1, ROWS, 1), lambda i: (i, 0, 0)),
        out_shape=jax.ShapeDtypeStruct((R // ROWS, ROWS, 1), jnp.int32),
        scratch_shapes=[pltpu.VMEM((ROWS, CW), jnp.float32)],
        compiler_params=pltpu.CompilerParams(
            dimension_semantics=("arbitrary",)),
    )(p2, rng3)
    return out.reshape(B, T)


# speedup vs baseline: 10.9702x; 10.9702x over previous
import functools

import jax
import jax.numpy as jnp
from jax.experimental import pallas as pl
from jax.experimental.pallas import tpu as pltpu

ROWS = 8
CW = 1024


def _scan_lanes(x, width, lanes):
    sh = 1
    while sh < width:
        rolled = pltpu.roll(x, sh, axis=1)
        x = x + jnp.where(lanes >= sh, rolled, 0.0)
        sh *= 2
    return x


def _sampler_body(nchunks, vsize, p_ref, rng_ref, out_ref, sums_ref):
    rngv = rng_ref[0]

    for c in range(nchunks):
        w = min(CW, vsize - c * CW)
        chunk = p_ref[:, c * CW:c * CW + w]
        sums_ref[:, c:c + 1] = jnp.sum(chunk, axis=1, keepdims=True)

    sums = sums_ref[...]
    clanes = jax.lax.broadcasted_iota(jnp.int32, (ROWS, 128), 1)
    incl = _scan_lanes(jnp.where(clanes < nchunks, sums, 0.0), 128, clanes)
    below = jnp.logical_and(incl < rngv, clanes < nchunks)
    widths = jnp.minimum(
        jnp.maximum(vsize - clanes * CW, 0), CW)
    base = jnp.sum(jnp.where(below, widths, 0), axis=1, keepdims=True)
    pstar = jnp.sum(jnp.where(below, sums, 0.0), axis=1, keepdims=True)
    cstar = jnp.sum(jnp.where(below, 1, 0), axis=1, keepdims=True)

    cand = jnp.zeros((ROWS, CW), jnp.float32)
    for c in range(nchunks):
        w = min(CW, vsize - c * CW)
        chunk = p_ref[:, c * CW:c * CW + w]
        if w < CW:
            chunk = jnp.concatenate(
                [chunk, jnp.zeros((ROWS, CW - w), jnp.float32)], axis=1)
        cand = jnp.where(cstar == c, chunk, cand)

    lanes = jax.lax.broadcasted_iota(jnp.int32, (ROWS, CW), 1)
    lc = _scan_lanes(cand, CW, lanes)
    cwidth = jnp.where(cstar >= nchunks, 0,
                       jnp.minimum(vsize - cstar * CW, CW))
    valid = jnp.logical_and(lanes < cwidth, pstar + lc < rngv)
    cnt = jnp.sum(jnp.where(valid, 1, 0), axis=1, keepdims=True)
    out_ref[0] = base + cnt


@jax.jit
def kernel(p, rng):
    B, T, V = p.shape
    R = B * T
    nchunks = -(-V // CW)
    p2 = p.reshape(R, V)
    rng3 = rng.reshape(R // ROWS, ROWS, 1)

    out = pl.pallas_call(
        functools.partial(_sampler_body, nchunks, V),
        grid=(R // ROWS,),
        in_specs=[
            pl.BlockSpec((ROWS, V), lambda i: (i, 0)),
            pl.BlockSpec((1, ROWS, 1), lambda i: (i, 0, 0)),
        ],
        out_specs=pl.BlockSpec((1, ROWS, 1), lambda i: (i, 0, 0)),
        out_shape=jax.ShapeDtypeStruct((R // ROWS, ROWS, 1), jnp.int32),
        scratch_shapes=[pltpu.VMEM((ROWS, 128), jnp.float32)],
        compiler_params=pltpu.CompilerParams(
            dimension_semantics=("arbitrary",)),
    )(p2, rng3)
    return out.reshape(B, T)


# speedup vs baseline: 14.9807x; 1.3656x over previous
import functools

import jax
import jax.numpy as jnp
from jax.experimental import pallas as pl
from jax.experimental.pallas import tpu as pltpu

ROWS = 16
CW = 1024


def _scan_lanes(x, width, lanes):
    sh = 1
    while sh < width:
        rolled = pltpu.roll(x, sh, axis=1)
        x = x + jnp.where(lanes >= sh, rolled, 0.0)
        sh *= 2
    return x


def _sampler_body(nchunks, vsize, p_ref, rng_ref, out_ref, sums_ref):
    rngv = rng_ref[0]

    for c in range(nchunks):
        w = min(CW, vsize - c * CW)
        chunk = p_ref[:, c * CW:c * CW + w]
        sums_ref[:, c:c + 1] = jnp.sum(chunk, axis=1, keepdims=True)

    sums = sums_ref[...]
    clanes = jax.lax.broadcasted_iota(jnp.int32, (ROWS, 128), 1)
    incl = _scan_lanes(jnp.where(clanes < nchunks, sums, 0.0), 128, clanes)
    below = jnp.logical_and(incl < rngv, clanes < nchunks)
    widths = jnp.minimum(
        jnp.maximum(vsize - clanes * CW, 0), CW)
    base = jnp.sum(jnp.where(below, widths, 0), axis=1, keepdims=True)
    pstar = jnp.sum(jnp.where(below, sums, 0.0), axis=1, keepdims=True)
    cstar = jnp.sum(jnp.where(below, 1, 0), axis=1, keepdims=True)

    cand = jnp.zeros((ROWS, CW), jnp.float32)
    for c in range(nchunks):
        w = min(CW, vsize - c * CW)
        chunk = p_ref[:, c * CW:c * CW + w]
        if w < CW:
            chunk = jnp.concatenate(
                [chunk, jnp.zeros((ROWS, CW - w), jnp.float32)], axis=1)
        cand = jnp.where(cstar == c, chunk, cand)

    lanes = jax.lax.broadcasted_iota(jnp.int32, (ROWS, CW), 1)
    lc = _scan_lanes(cand, CW, lanes)
    cwidth = jnp.where(cstar >= nchunks, 0,
                       jnp.minimum(vsize - cstar * CW, CW))
    valid = jnp.logical_and(lanes < cwidth, pstar + lc < rngv)
    cnt = jnp.sum(jnp.where(valid, 1, 0), axis=1, keepdims=True)
    out_ref[0] = base + cnt


@jax.jit
def kernel(p, rng):
    B, T, V = p.shape
    R = B * T
    nchunks = -(-V // CW)
    p2 = p.reshape(R, V)
    rng3 = rng.reshape(R // ROWS, ROWS, 1)

    out = pl.pallas_call(
        functools.partial(_sampler_body, nchunks, V),
        grid=(R // ROWS,),
        in_specs=[
            pl.BlockSpec((ROWS, V), lambda i: (i, 0)),
            pl.BlockSpec((1, ROWS, 1), lambda i: (i, 0, 0)),
        ],
        out_specs=pl.BlockSpec((1, ROWS, 1), lambda i: (i, 0, 0)),
        out_shape=jax.ShapeDtypeStruct((R // ROWS, ROWS, 1), jnp.int32),
        scratch_shapes=[pltpu.VMEM((ROWS, 128), jnp.float32)],
        compiler_params=pltpu.CompilerParams(
            dimension_semantics=("arbitrary",)),
    )(p2, rng3)
    return out.reshape(B, T)


# speedup vs baseline: 17.9918x; 1.2010x over previous
import functools

import jax
import jax.numpy as jnp
from jax.experimental import pallas as pl
from jax.experimental.pallas import tpu as pltpu

ROWS = 32
CW = 1024


def _scan_lanes(x, width, lanes):
    sh = 1
    while sh < width:
        rolled = pltpu.roll(x, sh, axis=1)
        x = x + jnp.where(lanes >= sh, rolled, 0.0)
        sh *= 2
    return x


def _sampler_body(nchunks, vsize, p_ref, rng_ref, out_ref, sums_ref):
    rngv = rng_ref[0]

    for c in range(nchunks):
        w = min(CW, vsize - c * CW)
        chunk = p_ref[:, c * CW:c * CW + w]
        sums_ref[:, c:c + 1] = jnp.sum(chunk, axis=1, keepdims=True)

    sums = sums_ref[...]
    clanes = jax.lax.broadcasted_iota(jnp.int32, (ROWS, 128), 1)
    incl = _scan_lanes(jnp.where(clanes < nchunks, sums, 0.0), 128, clanes)
    below = jnp.logical_and(incl < rngv, clanes < nchunks)
    widths = jnp.minimum(
        jnp.maximum(vsize - clanes * CW, 0), CW)
    base = jnp.sum(jnp.where(below, widths, 0), axis=1, keepdims=True)
    pstar = jnp.sum(jnp.where(below, sums, 0.0), axis=1, keepdims=True)
    cstar = jnp.sum(jnp.where(below, 1, 0), axis=1, keepdims=True)

    cand = jnp.zeros((ROWS, CW), jnp.float32)
    for c in range(nchunks):
        w = min(CW, vsize - c * CW)
        chunk = p_ref[:, c * CW:c * CW + w]
        if w < CW:
            chunk = jnp.concatenate(
                [chunk, jnp.zeros((ROWS, CW - w), jnp.float32)], axis=1)
        cand = jnp.where(cstar == c, chunk, cand)

    lanes = jax.lax.broadcasted_iota(jnp.int32, (ROWS, CW), 1)
    lc = _scan_lanes(cand, CW, lanes)
    cwidth = jnp.where(cstar >= nchunks, 0,
                       jnp.minimum(vsize - cstar * CW, CW))
    valid = jnp.logical_and(lanes < cwidth, pstar + lc < rngv)
    cnt = jnp.sum(jnp.where(valid, 1, 0), axis=1, keepdims=True)
    out_ref[0] = base + cnt


@jax.jit
def kernel(p, rng):
    B, T, V = p.shape
    R = B * T
    nchunks = -(-V // CW)
    p2 = p.reshape(R, V)
    rng3 = rng.reshape(R // ROWS, ROWS, 1)

    out = pl.pallas_call(
        functools.partial(_sampler_body, nchunks, V),
        grid=(R // ROWS,),
        in_specs=[
            pl.BlockSpec((ROWS, V), lambda i: (i, 0)),
            pl.BlockSpec((1, ROWS, 1), lambda i: (i, 0, 0)),
        ],
        out_specs=pl.BlockSpec((1, ROWS, 1), lambda i: (i, 0, 0)),
        out_shape=jax.ShapeDtypeStruct((R // ROWS, ROWS, 1), jnp.int32),
        scratch_shapes=[pltpu.VMEM((ROWS, 128), jnp.float32)],
        compiler_params=pltpu.CompilerParams(
            dimension_semantics=("arbitrary",)),
    )(p2, rng3)
    return out.reshape(B, T)
